# Initial kernel scaffold; baseline (speedup 1.0000x reference)
import jax
import jax.numpy as jnp
from jax.experimental import pallas as pl


def kernel(adj_mat_list, node_init, W1, b1, W2, b2):
    raise NotImplementedError("write your pallas kernel here")


# speedup vs baseline: 1.1191x; 1.1191x over previous
import functools

import jax
import jax.numpy as jnp
from jax.experimental import pallas as pl
from jax.experimental.pallas import tpu as pltpu

N = 4096
D = 128
R = 3
BM = 256


def _layer_body(adj_ref, x_ref, w_ref, b_ref, o_ref, xw_ref):
    @pl.when(pl.program_id(0) == 0)
    def _():
        for r in range(R):
            xw_ref[r] = jnp.dot(x_ref[...], w_ref[r],
                                preferred_element_type=jnp.float32)

    acc = jnp.dot(adj_ref[0], xw_ref[0], preferred_element_type=jnp.float32)
    for r in range(1, R):
        acc += jnp.dot(adj_ref[r], xw_ref[r],
                       preferred_element_type=jnp.float32)

    y = jnp.maximum(acc + b_ref[...], 0.0)
    nrm = jnp.sqrt(jnp.sum(y * y, axis=1, keepdims=True))
    o_ref[...] = y / jnp.maximum(nrm, 1e-12)


@functools.partial(jax.jit, static_argnames=())
def _layer(adj, x, w, b):
    return pl.pallas_call(
        _layer_body,
        grid=(N // BM,),
        in_specs=[
            pl.BlockSpec((R, BM, N), lambda m: (0, m, 0)),
            pl.BlockSpec((N, D), lambda m: (0, 0)),
            pl.BlockSpec((R, D, D), lambda m: (0, 0, 0)),
            pl.BlockSpec((1, D), lambda m: (0, 0)),
        ],
        out_specs=pl.BlockSpec((BM, D), lambda m: (m, 0)),
        out_shape=jax.ShapeDtypeStruct((N, D), jnp.float32),
        scratch_shapes=[pltpu.VMEM((R, N, D), jnp.float32)],
    )(adj, x, w, b)


def kernel(adj_mat_list, node_init, W1, b1, W2, b2):
    out = _layer(adj_mat_list, node_init, W1, b1.reshape(1, D))
    out = _layer(adj_mat_list, out, W2, b2.reshape(1, D))
    return out
